# baseline (speedup 1.0000x reference)
import jax
import jax.numpy as jnp
from jax.experimental import pallas as pl

S_BLK = 256


def _add_pe_kernel(x_ref, pe_ref, o_ref):
    o_ref[...] = x_ref[...] + pe_ref[...][None]


def kernel(input_seq, pe):
    B, S, D = input_seq.shape
    grid = (S // S_BLK,)
    return pl.pallas_call(
        _add_pe_kernel,
        grid=grid,
        in_specs=[
            pl.BlockSpec((B, S_BLK, D), lambda i: (0, i, 0)),
            pl.BlockSpec((S_BLK, D), lambda i: (i, 0)),
        ],
        out_specs=pl.BlockSpec((B, S_BLK, D), lambda i: (0, i, 0)),
        out_shape=jax.ShapeDtypeStruct((B, S, D), input_seq.dtype),
    )(input_seq, pe)


# speedup vs baseline: 1.0104x; 1.0104x over previous
import jax
import jax.numpy as jnp
from jax.experimental import pallas as pl

S_BLK = 512


def _add_pe_kernel(x_ref, pe_ref, o_ref):
    o_ref[...] = x_ref[...] + pe_ref[...][None]


def kernel(input_seq, pe):
    B, S, D = input_seq.shape
    grid = (S // S_BLK,)
    return pl.pallas_call(
        _add_pe_kernel,
        grid=grid,
        in_specs=[
            pl.BlockSpec((B, S_BLK, D), lambda i: (0, i, 0)),
            pl.BlockSpec((S_BLK, D), lambda i: (i, 0)),
        ],
        out_specs=pl.BlockSpec((B, S_BLK, D), lambda i: (0, i, 0)),
        out_shape=jax.ShapeDtypeStruct((B, S, D), input_seq.dtype),
    )(input_seq, pe)


# speedup vs baseline: 1.0305x; 1.0199x over previous
import jax
import jax.numpy as jnp
from jax.experimental import pallas as pl

S_BLK = 1024


def _add_pe_kernel(x_ref, pe_ref, o_ref):
    o_ref[...] = x_ref[...] + pe_ref[...][None]


def kernel(input_seq, pe):
    B, S, D = input_seq.shape
    B_BLK = 2
    grid = (S // S_BLK, B // B_BLK)
    return pl.pallas_call(
        _add_pe_kernel,
        grid=grid,
        in_specs=[
            pl.BlockSpec((B_BLK, S_BLK, D), lambda i, b: (b, i, 0)),
            pl.BlockSpec((S_BLK, D), lambda i, b: (i, 0)),
        ],
        out_specs=pl.BlockSpec((B_BLK, S_BLK, D), lambda i, b: (b, i, 0)),
        out_shape=jax.ShapeDtypeStruct((B, S, D), input_seq.dtype),
    )(input_seq, pe)


# speedup vs baseline: 1.0348x; 1.0042x over previous
import jax
import jax.numpy as jnp
from jax.experimental import pallas as pl

S_BLK = 2048


def _add_pe_kernel(x_ref, pe_ref, o_ref):
    o_ref[...] = x_ref[...] + pe_ref[...][None]


def kernel(input_seq, pe):
    B, S, D = input_seq.shape
    B_BLK = 1
    grid = (S // S_BLK, B // B_BLK)
    return pl.pallas_call(
        _add_pe_kernel,
        grid=grid,
        in_specs=[
            pl.BlockSpec((B_BLK, S_BLK, D), lambda i, b: (b, i, 0)),
            pl.BlockSpec((S_BLK, D), lambda i, b: (i, 0)),
        ],
        out_specs=pl.BlockSpec((B_BLK, S_BLK, D), lambda i, b: (b, i, 0)),
        out_shape=jax.ShapeDtypeStruct((B, S, D), input_seq.dtype),
    )(input_seq, pe)
